# baseline (speedup 1.0000x reference)
import functools

import jax
import jax.numpy as jnp
from jax import lax
from jax.experimental import pallas as pl
from jax.experimental.pallas import tpu as pltpu
from jax.experimental.pallas import tpu_sc as plsc

BATCH = 16384
HIST = 50
DIM = 16
TOTAL = BATCH * HIST

_info = plsc.get_sparse_core_info()
NC, NS = _info.num_cores, _info.num_subcores
NW = NC * NS
BBLK = 128
NBT = BATCH // BBLK // NW
IDXB = BBLK * HIST
NU = HIST // 2

P_SHAPE = (HIST, DIM // 8, BATCH // BBLK, 8 * BBLK)


NTC = 1000000 // BBLK
TAIL = 1000000 - NTC * BBLK


def _make_detile():
    mesh = plsc.VectorSubcoreMesh(core_axis_name="c", subcore_axis_name="s")

    @functools.partial(
        pl.kernel,
        out_type=jax.ShapeDtypeStruct((16 * 1000000,), jnp.float32),
        mesh=mesh,
        scratch_types=[
            pltpu.VMEM((DIM, BBLK), jnp.float32),
            pltpu.VMEM((DIM, BBLK), jnp.float32),
            pltpu.VMEM((BBLK * DIM,), jnp.float32),
            pltpu.VMEM((BBLK * DIM,), jnp.float32),
            pltpu.SemaphoreType.DMA,
            pltpu.SemaphoreType.DMA,
            pltpu.SemaphoreType.DMA,
        ],
        compiler_params=pltpu.CompilerParams(
            use_tc_tiling_on_sc=True, needs_layout_passes=False
        ),
    )
    def detile(
        embT_hbm, tailT_hbm, out_hbm, col0, col1, lin0, lin1, sem_g, sem_s0, sem_s1
    ):
        wid = lax.axis_index("s") * NC + lax.axis_index("c")
        lane = lax.iota(jnp.int32, 16)
        cols = (col0, col1)
        lins = (lin0, lin1)
        sems = (sem_s0, sem_s1)
        nfull = NTC // NW

        def fire_load(c, buf):
            pltpu.async_copy(embT_hbm.at[:, pl.ds(c * BBLK, BBLK)], buf, sem_g)

        def wait_load(c, buf):
            pltpu.make_async_copy(
                embT_hbm.at[:, pl.ds(c * BBLK, BBLK)], buf, sem_g
            ).wait()

        def transpose_col(C, L):
            @pl.loop(0, DIM, unroll=4)
            def _(f):
                for g in range(BBLK // 16):
                    v = C[f, pl.ds(g * 16, 16)]
                    plsc.store_scatter(L, [(g * 16 + lane) * DIM + f], v)

        def col_of(t):
            return t * NW + wid

        fire_load(col_of(0), cols[0])

        @pl.loop(0, nfull, step=2)
        def _(t):
            for p in range(2):
                tt = t + p
                c = col_of(tt)
                wait_load(c, cols[p])

                @pl.when(tt + 1 < nfull)
                def _():
                    fire_load(col_of(tt + 1), cols[1 - p])

                @pl.when(tt >= 2)
                def _():
                    pltpu.make_async_copy(
                        lins[p], out_hbm.at[pl.ds(0, BBLK * DIM)], sems[p]
                    ).wait()

                transpose_col(cols[p], lins[p])
                pltpu.async_copy(
                    lins[p], out_hbm.at[pl.ds(c * BBLK * DIM, BBLK * DIM)], sems[p]
                )

        for p in range(2):
            pltpu.make_async_copy(
                lins[p], out_hbm.at[pl.ds(0, BBLK * DIM)], sems[p]
            ).wait()

        rem = NTC - nfull * NW

        @pl.when(wid < rem)
        def _():
            c = nfull * NW + wid
            pltpu.sync_copy(embT_hbm.at[:, pl.ds(c * BBLK, BBLK)], cols[0])
            transpose_col(cols[0], lins[0])
            pltpu.sync_copy(lins[0], out_hbm.at[pl.ds(c * BBLK * DIM, BBLK * DIM)])

        @pl.when(wid == rem)
        def _():
            pltpu.sync_copy(tailT_hbm, cols[0])
            transpose_col(cols[0], lins[0])
            pltpu.sync_copy(
                lins[0].at[pl.ds(0, TAIL * DIM)],
                out_hbm.at[pl.ds(NTC * BBLK * DIM, TAIL * DIM)],
            )

    return detile


HHALF = HIST // 2
SUB = HHALF * BBLK // 2


def _make_gather():
    mesh = plsc.VectorSubcoreMesh(core_axis_name="c", subcore_axis_name="s")

    @functools.partial(
        pl.kernel,
        out_type=jax.ShapeDtypeStruct(P_SHAPE, jnp.float32),
        mesh=mesh,
        scratch_types=[
            pltpu.VMEM((IDXB,), jnp.int32),
            pltpu.VMEM((IDXB,), jnp.int32),
            pltpu.VMEM((SUB, DIM), jnp.float32),
            pltpu.VMEM((SUB, DIM), jnp.float32),
            pltpu.VMEM((2, HHALF, 8 * BBLK), jnp.float32),
            pltpu.SemaphoreType.DMA,
            pltpu.SemaphoreType.DMA,
            pltpu.SemaphoreType.DMA,
        ],
        compiler_params=pltpu.CompilerParams(
            use_tc_tiling_on_sc=False, needs_layout_passes=False
        ),
    )
    def gather(
        idx_hbm,
        table_hbm,
        out_hbm,
        idx_v,
        idxT_v,
        rows0,
        rows1,
        tall,
        sem_g0,
        sem_g1,
        sem_s,
    ):
        wid = lax.axis_index("s") * NC + lax.axis_index("c")
        lane = lax.iota(jnp.int32, 16)
        ft_ids = lane // 8
        fi_off = (lane % 8) * BBLK
        rows_bufs = (rows0, rows1)

        sem_gs = (sem_g0, sem_g1)

        def fire_gather(bt, half, c, rbuf, sem):
            off = half * 2 * SUB + c * SUB
            pltpu.async_copy(table_hbm.at[idxT_v.at[pl.ds(off, SUB)]], rbuf, sem)

        def wait_gather(rbuf, sem):
            pltpu.make_async_copy(
                table_hbm.at[idxT_v.at[pl.ds(0, SUB)]], rbuf, sem
            ).wait()

        def fire_stores(bt, half):
            for ft in range(2):
                pltpu.async_copy(
                    tall.at[ft],
                    out_hbm.at[pl.ds(half * HHALF, HHALF), ft, bt],
                    sem_s,
                )

        def wait_stores(bt):
            for ft in range(2):
                pltpu.make_async_copy(
                    tall.at[ft], out_hbm.at[pl.ds(0, HHALF), ft, bt], sem_s
                ).wait()

        def transpose_sub(R, c):
            @pl.loop(0, SUB, unroll=8)
            def _(k):
                j2 = c * SUB + k
                hh = j2 >> 7
                plsc.store_scatter(
                    tall,
                    [ft_ids, jnp.full((16,), hh, jnp.int32), fi_off + (j2 & 127)],
                    R[k, :],
                )

        for r in range(NBT):
            bt = wid * NBT + r
            pltpu.sync_copy(idx_hbm.at[pl.ds(bt * IDXB, IDXB)], idx_v)

            @pl.loop(0, HIST)
            def _(h):
                for g in range(BBLK // 16):
                    v = plsc.load_gather(idx_v, [(g * 16 + lane) * HIST + h])
                    idxT_v[pl.ds(h * BBLK + g * 16, 16)] = v

            fire_gather(bt, 0, 0, rows_bufs[0], sem_gs[0])
            fire_gather(bt, 0, 1, rows_bufs[1], sem_gs[1])

            for half in range(2):
                if not (r == 0 and half == 0):
                    wait_stores(bt)
                for c in range(2):
                    wait_gather(rows_bufs[c], sem_gs[c])
                    transpose_sub(rows_bufs[c], c)
                    if half == 0:
                        fire_gather(bt, 1, c, rows_bufs[c], sem_gs[c])
                fire_stores(bt, half)

        wait_stores(wid * NBT)

    return gather


_detile = _make_detile()
_gather = _make_gather()


def kernel(inputs, emb):
    emb_t = emb.T
    tail_t = jnp.pad(emb_t[:, NTC * BBLK :], ((0, 0), (0, BBLK - TAIL)))
    emb_lin = _detile(emb_t, tail_t)
    p = _gather(inputs.reshape(TOTAL), emb_lin.reshape(1000000, DIM))
    p5 = p.reshape(HIST, DIM // 8, BATCH // BBLK, 8, BBLK)
    return p5.transpose(2, 4, 0, 1, 3).reshape(BATCH, HIST, DIM)


# speedup vs baseline: 1.4353x; 1.4353x over previous
import functools

import jax
import jax.numpy as jnp
from jax import lax
from jax.experimental import pallas as pl
from jax.experimental.pallas import tpu as pltpu
from jax.experimental.pallas import tpu_sc as plsc

BATCH = 16384
HIST = 50
DIM = 16
TOTAL = BATCH * HIST

_info = plsc.get_sparse_core_info()
NC, NS = _info.num_cores, _info.num_subcores
NW = NC * NS
BBLK = 128
NBT = BATCH // BBLK // NW
IDXB = BBLK * HIST
NU = HIST // 2

P_SHAPE = (HIST, DIM // 8, BATCH // BBLK, 8, BBLK)
TPAD = BBLK + 1


NTC = 1000000 // BBLK
TAIL = 1000000 - NTC * BBLK


def _make_detile():
    mesh = plsc.VectorSubcoreMesh(core_axis_name="c", subcore_axis_name="s")

    @functools.partial(
        pl.kernel,
        out_type=jax.ShapeDtypeStruct((16 * 1000000,), jnp.float32),
        mesh=mesh,
        scratch_types=[
            pltpu.VMEM((DIM, BBLK), jnp.float32),
            pltpu.VMEM((DIM, BBLK), jnp.float32),
            pltpu.VMEM((BBLK * DIM,), jnp.float32),
            pltpu.VMEM((BBLK * DIM,), jnp.float32),
            pltpu.SemaphoreType.DMA,
            pltpu.SemaphoreType.DMA,
            pltpu.SemaphoreType.DMA,
        ],
        compiler_params=pltpu.CompilerParams(
            use_tc_tiling_on_sc=True, needs_layout_passes=False
        ),
    )
    def detile(
        embT_hbm, tailT_hbm, out_hbm, col0, col1, lin0, lin1, sem_g, sem_s0, sem_s1
    ):
        wid = lax.axis_index("s") * NC + lax.axis_index("c")
        lane = lax.iota(jnp.int32, 16)
        cols = (col0, col1)
        lins = (lin0, lin1)
        sems = (sem_s0, sem_s1)
        nfull = NTC // NW

        def fire_load(c, buf):
            pltpu.async_copy(embT_hbm.at[:, pl.ds(c * BBLK, BBLK)], buf, sem_g)

        def wait_load(c, buf):
            pltpu.make_async_copy(
                embT_hbm.at[:, pl.ds(c * BBLK, BBLK)], buf, sem_g
            ).wait()

        def transpose_col(C, L):
            @pl.loop(0, DIM, unroll=4)
            def _(f):
                for g in range(BBLK // 16):
                    v = C[f, pl.ds(g * 16, 16)]
                    plsc.store_scatter(L, [(g * 16 + lane) * DIM + f], v)

        def col_of(t):
            return t * NW + wid

        fire_load(col_of(0), cols[0])

        @pl.loop(0, nfull, step=2)
        def _(t):
            for p in range(2):
                tt = t + p
                c = col_of(tt)
                wait_load(c, cols[p])

                @pl.when(tt + 1 < nfull)
                def _():
                    fire_load(col_of(tt + 1), cols[1 - p])

                @pl.when(tt >= 2)
                def _():
                    pltpu.make_async_copy(
                        lins[p], out_hbm.at[pl.ds(0, BBLK * DIM)], sems[p]
                    ).wait()

                transpose_col(cols[p], lins[p])
                pltpu.async_copy(
                    lins[p], out_hbm.at[pl.ds(c * BBLK * DIM, BBLK * DIM)], sems[p]
                )

        for p in range(2):
            pltpu.make_async_copy(
                lins[p], out_hbm.at[pl.ds(0, BBLK * DIM)], sems[p]
            ).wait()

        rem = NTC - nfull * NW

        @pl.when(wid < rem)
        def _():
            c = nfull * NW + wid
            pltpu.sync_copy(embT_hbm.at[:, pl.ds(c * BBLK, BBLK)], cols[0])
            transpose_col(cols[0], lins[0])
            pltpu.sync_copy(lins[0], out_hbm.at[pl.ds(c * BBLK * DIM, BBLK * DIM)])

        @pl.when(wid == rem)
        def _():
            pltpu.sync_copy(tailT_hbm, cols[0])
            transpose_col(cols[0], lins[0])
            pltpu.sync_copy(
                lins[0].at[pl.ds(0, TAIL * DIM)],
                out_hbm.at[pl.ds(NTC * BBLK * DIM, TAIL * DIM)],
            )

    return detile


HHALF = HIST // 2
SUB = HHALF * BBLK // 2


def _make_gather():
    mesh = plsc.VectorSubcoreMesh(core_axis_name="c", subcore_axis_name="s")

    @functools.partial(
        pl.kernel,
        out_type=jax.ShapeDtypeStruct(P_SHAPE, jnp.float32),
        mesh=mesh,
        scratch_types=[
            pltpu.VMEM((IDXB,), jnp.int32),
            pltpu.VMEM((IDXB,), jnp.int32),
            pltpu.VMEM((SUB, DIM), jnp.float32),
            pltpu.VMEM((SUB, DIM), jnp.float32),
            pltpu.VMEM((2, HHALF, 8, TPAD), jnp.float32),
            pltpu.SemaphoreType.DMA,
            pltpu.SemaphoreType.DMA,
            pltpu.SemaphoreType.DMA,
        ],
        compiler_params=pltpu.CompilerParams(
            use_tc_tiling_on_sc=False, needs_layout_passes=False
        ),
    )
    def gather(
        idx_hbm,
        table_hbm,
        out_hbm,
        idx_v,
        idxT_v,
        rows0,
        rows1,
        tall,
        sem_g0,
        sem_g1,
        sem_s,
    ):
        wid = lax.axis_index("s") * NC + lax.axis_index("c")
        lane = lax.iota(jnp.int32, 16)
        ft_ids = lane // 8
        fi_ids = lane % 8
        rows_bufs = (rows0, rows1)

        sem_gs = (sem_g0, sem_g1)

        def fire_gather(bt, half, c, rbuf, sem):
            off = half * 2 * SUB + c * SUB
            pltpu.async_copy(table_hbm.at[idxT_v.at[pl.ds(off, SUB)]], rbuf, sem)

        def wait_gather(rbuf, sem):
            pltpu.make_async_copy(
                table_hbm.at[idxT_v.at[pl.ds(0, SUB)]], rbuf, sem
            ).wait()

        def fire_stores(bt, half):
            for ft in range(2):
                pltpu.async_copy(
                    tall.at[ft, :, :, pl.ds(0, BBLK)],
                    out_hbm.at[pl.ds(half * HHALF, HHALF), ft, bt],
                    sem_s,
                )

        def wait_stores(bt):
            for ft in range(2):
                pltpu.make_async_copy(
                    tall.at[ft, :, :, pl.ds(0, BBLK)],
                    out_hbm.at[pl.ds(0, HHALF), ft, bt],
                    sem_s,
                ).wait()

        def transpose_sub(R, c):
            @pl.loop(0, SUB, unroll=8)
            def _(k):
                j2 = c * SUB + k
                hh = j2 >> 7
                plsc.store_scatter(
                    tall,
                    [
                        ft_ids,
                        jnp.full((16,), hh, jnp.int32),
                        fi_ids,
                        jnp.full((16,), j2 & 127, jnp.int32),
                    ],
                    R[k, :],
                )

        for r in range(NBT):
            bt = wid * NBT + r
            pltpu.sync_copy(idx_hbm.at[pl.ds(bt * IDXB, IDXB)], idx_v)

            @pl.loop(0, HIST)
            def _(h):
                for g in range(BBLK // 16):
                    v = plsc.load_gather(idx_v, [(g * 16 + lane) * HIST + h])
                    idxT_v[pl.ds(h * BBLK + g * 16, 16)] = v

            fire_gather(bt, 0, 0, rows_bufs[0], sem_gs[0])
            fire_gather(bt, 0, 1, rows_bufs[1], sem_gs[1])

            for half in range(2):
                if not (r == 0 and half == 0):
                    wait_stores(bt)
                for c in range(2):
                    wait_gather(rows_bufs[c], sem_gs[c])
                    transpose_sub(rows_bufs[c], c)
                    if half == 0:
                        fire_gather(bt, 1, c, rows_bufs[c], sem_gs[c])
                fire_stores(bt, half)

        wait_stores(wid * NBT)

    return gather


_detile = _make_detile()
_gather = _make_gather()


def kernel(inputs, emb):
    emb_t = emb.T
    tail_t = jnp.pad(emb_t[:, NTC * BBLK :], ((0, 0), (0, BBLK - TAIL)))
    emb_lin = _detile(emb_t, tail_t)
    p5 = _gather(inputs.reshape(TOTAL), emb_lin.reshape(1000000, DIM))
    return p5.transpose(2, 4, 0, 1, 3).reshape(BATCH, HIST, DIM)
